# Initial kernel scaffold; baseline (speedup 1.0000x reference)
import jax
import jax.numpy as jnp
from jax.experimental import pallas as pl


def kernel(x, region_percentiles, proxy_percentiles):
    raise NotImplementedError("write your pallas kernel here")


# speedup vs baseline: 3.5696x; 3.5696x over previous
import functools

import jax
import jax.numpy as jnp
from jax import lax
from jax.experimental import pallas as pl
from jax.experimental.pallas import tpu as pltpu
from jax.experimental.pallas import tpu_sc as plsc

REGIONS = 8
NC, NS, L = 2, 16, 16
NW = NC * NS
C_TOTAL = 96
CPW = C_TOTAL // NW
N_PIX = 224 * 224
NVEC = N_PIX // L


def _body(x_hbm, par_hbm, out_hbm, buf, par_v, sem):
    cid = lax.axis_index("c")
    sid = lax.axis_index("s")
    wid = sid * NC + cid
    iota = lax.iota(jnp.int32, L)
    inf = jnp.float32(jnp.inf)

    for j in range(CPW):
        ch = wid * CPW + j
        pltpu.sync_copy(x_hbm.at[ch], buf)
        pltpu.sync_copy(par_hbm.at[ch], par_v)

        def mm_step(i, carry):
            mn_v, mx_v = carry
            v = buf[pl.ds(i * L, L)]
            return jnp.minimum(mn_v, v), jnp.maximum(mx_v, v)

        mn_v, mx_v = lax.fori_loop(
            0, NVEC, mm_step,
            (jnp.full((L,), inf, jnp.float32), jnp.full((L,), -inf, jnp.float32)),
        )
        mn = jnp.min(mn_v)
        mx = jnp.max(mx_v)

        par = par_v[...]
        pos = jnp.where(iota < REGIONS - 1, par * (mx - mn) + mn, inf)
        s = lax.sort(pos)

        s_sc = [jnp.min(jnp.where(iota == i, s, inf)) for i in range(REGIONS - 1)]
        pp_sc = [jnp.min(jnp.where(iota == 8 + r, par, inf)) for r in range(REGIONS)]
        lefts = [mn] + s_sc
        rights = s_sc + [mx + jnp.float32(1e-6)]
        rv = [lefts[r] + pp_sc[r] * (rights[r] - lefts[r]) for r in range(REGIONS)]

        def map_step(i, acc):
            v = buf[pl.ds(i * L, L)]
            o = jnp.full((L,), rv[0], jnp.float32)
            for r in range(REGIONS - 1):
                o = jnp.where(v >= s_sc[r], rv[r + 1], o)
            buf[pl.ds(i * L, L)] = o
            return acc

        lax.fori_loop(0, NVEC, map_step, jnp.int32(0))
        pltpu.sync_copy(buf, out_hbm.at[ch])


@jax.jit
def kernel(x, region_percentiles, proxy_percentiles):
    B, c, H, W = x.shape
    xf = x.reshape(C_TOTAL, N_PIX)
    par = jnp.concatenate(
        [
            region_percentiles.reshape(C_TOTAL, REGIONS - 1),
            jnp.zeros((C_TOTAL, 1), jnp.float32),
            proxy_percentiles.reshape(C_TOTAL, REGIONS),
        ],
        axis=1,
    )

    mesh = plsc.VectorSubcoreMesh(core_axis_name="c", subcore_axis_name="s")
    out = pl.kernel(
        _body,
        out_type=jax.ShapeDtypeStruct((C_TOTAL, N_PIX), jnp.float32),
        mesh=mesh,
        compiler_params=pltpu.CompilerParams(needs_layout_passes=False),
        scratch_types=[
            pltpu.VMEM((N_PIX,), jnp.float32),
            pltpu.VMEM((L,), jnp.float32),
            pltpu.SemaphoreType.DMA,
        ],
    )(xf, par)
    return out.reshape(B, c, H, W)


# speedup vs baseline: 6.7260x; 1.8843x over previous
import functools

import jax
import jax.numpy as jnp
from jax import lax
from jax.experimental import pallas as pl
from jax.experimental.pallas import tpu as pltpu
from jax.experimental.pallas import tpu_sc as plsc

REGIONS = 8
NC, NS, L = 2, 16, 16
NW = NC * NS
C_TOTAL = 96
CPW = C_TOTAL // NW
N_PIX = 224 * 224
NVEC = N_PIX // L
MM_U = 8
MAP_U = 4


def _body(x_hbm, par_hbm, out_hbm, buf0, buf1, parb, is0, is1, os0, os1):
    cid = lax.axis_index("c")
    sid = lax.axis_index("s")
    wid = sid * NC + cid
    iota = lax.iota(jnp.int32, L)
    inf = jnp.float32(jnp.inf)

    ch0 = wid * CPW
    bufs = (buf0, buf1)
    isems = (is0, is1)
    osems = (os0, os1)
    pltpu.sync_copy(par_hbm.at[wid], parb)
    pltpu.async_copy(x_hbm.at[ch0], buf0, is0)
    pltpu.async_copy(x_hbm.at[ch0 + 1], buf1, is1)

    for j in range(CPW):
        b = j % 2
        ch = ch0 + j
        buf = bufs[b]
        pltpu.make_async_copy(x_hbm.at[ch], buf, isems[b]).wait()

        init = tuple(jnp.full((L,), inf, jnp.float32) for _ in range(MM_U)) + tuple(
            jnp.full((L,), -inf, jnp.float32) for _ in range(MM_U)
        )

        @plsc.parallel_loop(0, NVEC // MM_U, carry=init)
        def mm_loop(i, carry):
            mns = list(carry[:MM_U])
            mxs = list(carry[MM_U:])
            base = i * (MM_U * L)
            for k in range(MM_U):
                v = buf[pl.ds(base + k * L, L)]
                mns[k] = jnp.minimum(mns[k], v)
                mxs[k] = jnp.maximum(mxs[k], v)
            return tuple(mns) + tuple(mxs)

        mns = list(mm_loop[:MM_U])
        mxs = list(mm_loop[MM_U:])
        while len(mns) > 1:
            mns = [jnp.minimum(a, b) for a, b in zip(mns[::2], mns[1::2])]
            mxs = [jnp.maximum(a, b) for a, b in zip(mxs[::2], mxs[1::2])]
        mn = jnp.min(mns[0])
        mx = jnp.max(mxs[0])

        par = parb[pl.ds(j * L, L)]
        pos = jnp.where(iota < REGIONS - 1, par * (mx - mn) + mn, inf)
        s = lax.sort(pos)

        s_sc = [jnp.min(jnp.where(iota == i, s, inf)) for i in range(REGIONS - 1)]
        pp_sc = [jnp.min(jnp.where(iota == 8 + r, par, inf)) for r in range(REGIONS)]
        lefts = [mn] + s_sc
        rights = s_sc + [mx + jnp.float32(1e-6)]
        rv = [lefts[r] + pp_sc[r] * (rights[r] - lefts[r]) for r in range(REGIONS)]

        if 0 < j < CPW - 1:
            b2 = (j + 1) % 2
            pltpu.make_async_copy(bufs[b2], out_hbm.at[ch - 1], osems[b2]).wait()
            pltpu.async_copy(x_hbm.at[ch + 1], bufs[b2], isems[b2])

        @plsc.parallel_loop(0, NVEC // MAP_U)
        def map_loop(i):
            base = i * (MAP_U * L)
            for k in range(MAP_U):
                v = buf[pl.ds(base + k * L, L)]
                o = jnp.full((L,), rv[0], jnp.float32)
                for r in range(REGIONS - 1):
                    o = jnp.where(v >= s_sc[r], rv[r + 1], o)
                buf[pl.ds(base + k * L, L)] = o

        pltpu.async_copy(buf, out_hbm.at[ch], osems[b])

    pltpu.make_async_copy(bufs[(CPW - 2) % 2], out_hbm.at[ch0 + CPW - 2], osems[(CPW - 2) % 2]).wait()
    pltpu.make_async_copy(bufs[(CPW - 1) % 2], out_hbm.at[ch0 + CPW - 1], osems[(CPW - 1) % 2]).wait()


@jax.jit
def kernel(x, region_percentiles, proxy_percentiles):
    B, c, H, W = x.shape
    xf = x.reshape(C_TOTAL, N_PIX)
    par = jnp.concatenate(
        [
            region_percentiles.reshape(C_TOTAL, REGIONS - 1),
            jnp.zeros((C_TOTAL, 1), jnp.float32),
            proxy_percentiles.reshape(C_TOTAL, REGIONS),
        ],
        axis=1,
    ).reshape(NW, CPW * L)

    mesh = plsc.VectorSubcoreMesh(core_axis_name="c", subcore_axis_name="s")
    out = pl.kernel(
        _body,
        out_type=jax.ShapeDtypeStruct((C_TOTAL, N_PIX), jnp.float32),
        mesh=mesh,
        compiler_params=pltpu.CompilerParams(needs_layout_passes=False),
        scratch_types=[
            pltpu.VMEM((N_PIX,), jnp.float32),
            pltpu.VMEM((N_PIX,), jnp.float32),
            pltpu.VMEM((CPW * L,), jnp.float32),
            pltpu.SemaphoreType.DMA,
            pltpu.SemaphoreType.DMA,
            pltpu.SemaphoreType.DMA,
            pltpu.SemaphoreType.DMA,
        ],
    )(xf, par)
    return out.reshape(B, c, H, W)


# speedup vs baseline: 11.3980x; 1.6946x over previous
import jax
import jax.numpy as jnp
from jax import lax
from jax.experimental import pallas as pl
from jax.experimental.pallas import tpu as pltpu
from jax.experimental.pallas import tpu_sc as plsc

REGIONS = 8
NC, NS, L = 2, 16, 16
NW = NC * NS
B, CCH, H, W = 32, 3, 224, 224
C_TOTAL = B * CCH
CPW = C_TOTAL // NW
WVEC = W // L
MM_U = 8


def _body(x_hbm, par_hbm, out_hbm, buf0, buf1, parb, is0, is1, os0, os1):
    cid = lax.axis_index("c")
    sid = lax.axis_index("s")
    wid = sid * NC + cid
    iota = lax.iota(jnp.int32, L)
    inf = jnp.float32(jnp.inf)

    ch0 = wid * CPW
    bufs = (buf0, buf1)
    isems = (is0, is1)
    osems = (os0, os1)

    def plane(ref, ch):
        return ref.at[ch // CCH, ch % CCH]

    pltpu.sync_copy(par_hbm.at[wid], parb)
    pltpu.async_copy(plane(x_hbm, ch0), buf0, is0)
    pltpu.async_copy(plane(x_hbm, ch0 + 1), buf1, is1)

    for j in range(CPW):
        b = j % 2
        ch = ch0 + j
        buf = bufs[b]
        pltpu.make_async_copy(plane(x_hbm, ch), buf, isems[b]).wait()

        init = tuple(jnp.full((L,), inf, jnp.float32) for _ in range(MM_U)) + tuple(
            jnp.full((L,), -inf, jnp.float32) for _ in range(MM_U)
        )

        @plsc.parallel_loop(0, H, carry=init)
        def mm_loop(i, carry):
            mns = list(carry[:MM_U])
            mxs = list(carry[MM_U:])
            for k in range(WVEC):
                v = buf[i, pl.ds(k * L, L)]
                mns[k % MM_U] = jnp.minimum(mns[k % MM_U], v)
                mxs[k % MM_U] = jnp.maximum(mxs[k % MM_U], v)
            return tuple(mns) + tuple(mxs)

        mns = list(mm_loop[:MM_U])
        mxs = list(mm_loop[MM_U:])
        while len(mns) > 1:
            mns = [jnp.minimum(a, b2_) for a, b2_ in zip(mns[::2], mns[1::2])]
            mxs = [jnp.maximum(a, b2_) for a, b2_ in zip(mxs[::2], mxs[1::2])]
        mn = jnp.min(mns[0])
        mx = jnp.max(mxs[0])

        par = parb[pl.ds(j * L, L)]
        pos = jnp.where(iota < REGIONS - 1, par * (mx - mn) + mn, inf)
        s = lax.sort(pos)

        s_sc = [jnp.min(jnp.where(iota == i, s, inf)) for i in range(REGIONS - 1)]
        pp_sc = [jnp.min(jnp.where(iota == 8 + r, par, inf)) for r in range(REGIONS)]
        lefts = [mn] + s_sc
        rights = s_sc + [mx + jnp.float32(1e-6)]
        rv = [lefts[r] + pp_sc[r] * (rights[r] - lefts[r]) for r in range(REGIONS)]

        if 0 < j < CPW - 1:
            b2 = (j + 1) % 2
            pltpu.make_async_copy(bufs[b2], plane(out_hbm, ch - 1), osems[b2]).wait()
            pltpu.async_copy(plane(x_hbm, ch + 1), bufs[b2], isems[b2])

        @plsc.parallel_loop(0, H)
        def map_loop(i):
            for k in range(WVEC):
                v = buf[i, pl.ds(k * L, L)]
                o = jnp.full((L,), rv[0], jnp.float32)
                for r in range(REGIONS - 1):
                    o = jnp.where(v >= s_sc[r], rv[r + 1], o)
                buf[i, pl.ds(k * L, L)] = o

        pltpu.async_copy(buf, plane(out_hbm, ch), osems[b])

    pltpu.make_async_copy(bufs[(CPW - 2) % 2], plane(out_hbm, ch0 + CPW - 2), osems[(CPW - 2) % 2]).wait()
    pltpu.make_async_copy(bufs[(CPW - 1) % 2], plane(out_hbm, ch0 + CPW - 1), osems[(CPW - 1) % 2]).wait()


@jax.jit
def kernel(x, region_percentiles, proxy_percentiles):
    par = jnp.concatenate(
        [
            region_percentiles.reshape(C_TOTAL, REGIONS - 1),
            jnp.zeros((C_TOTAL, 1), jnp.float32),
            proxy_percentiles.reshape(C_TOTAL, REGIONS),
        ],
        axis=1,
    ).reshape(NW, CPW * L)

    mesh = plsc.VectorSubcoreMesh(core_axis_name="c", subcore_axis_name="s")
    out = pl.kernel(
        _body,
        out_type=jax.ShapeDtypeStruct((B, CCH, H, W), jnp.float32),
        mesh=mesh,
        compiler_params=pltpu.CompilerParams(needs_layout_passes=False),
        scratch_types=[
            pltpu.VMEM((H, W), jnp.float32),
            pltpu.VMEM((H, W), jnp.float32),
            pltpu.VMEM((CPW * L,), jnp.float32),
            pltpu.SemaphoreType.DMA,
            pltpu.SemaphoreType.DMA,
            pltpu.SemaphoreType.DMA,
            pltpu.SemaphoreType.DMA,
        ],
    )(x, par)
    return out
